# baseline (speedup 1.0000x reference)
import functools

import jax
import jax.numpy as jnp
from jax.experimental import pallas as pl
from jax.experimental.pallas import tpu as pltpu

_S = 4096
_K = 12
_CB = 128


def _topk_weights(p, iota):
    def body(_, carry):
        pv, accw = carry
        mx = jnp.max(pv, axis=1, keepdims=True)
        eq = pv == mx
        first = jnp.min(jnp.where(eq, iota, _S), axis=1, keepdims=True)
        sel = iota == first
        accw = accw + sel.astype(jnp.float32)
        pv = jnp.where(sel, -jnp.inf, pv)
        return pv, accw

    _, accw = jax.lax.fori_loop(
        0, _K, body, (p, jnp.zeros_like(p)), unroll=True
    )
    return accw


def _weights_kernel(out_ref, tau_ref, w_ref):
    o0 = out_ref[:, 0, :]
    o1 = out_ref[:, 1, :]
    m = jnp.maximum(o0, o1)
    e0 = jnp.exp(o0 - m)
    e1 = jnp.exp(o1 - m)
    s = e0 + e1
    pf = e1 / s
    pb = e0 / s

    tau = tau_ref[0, 0]
    fg_thres = jax.nn.sigmoid(tau)
    bg_thres = 1.0 - fg_thres

    iota = jax.lax.broadcasted_iota(jnp.int32, pf.shape, 1)
    inv_k = jnp.float32(1.0 / _K)

    for cls, p, thres in ((0, pf, fg_thres), (1, pb, bg_thres)):
        mask = (p > thres).astype(jnp.float32)
        cnt = jnp.sum(mask, axis=1, keepdims=True)
        tk = _topk_weights(p, iota)
        safe = jnp.where(cnt > 0, cnt, jnp.float32(1.0))
        w = jnp.where(cnt > 0, mask / safe, tk * inv_k)
        w_ref[:, cls, :] = w


def _pool_kernel(feat_ref, w_ref, out_ref):
    feat = feat_ref[0]
    w = w_ref[0]
    out_ref[0] = jax.lax.dot_general(
        w, feat, (((1,), (1,)), ((), ())),
        preferred_element_type=jnp.float32,
    )


@jax.jit
def _run(feature_q, out, tau):
    bs, C = feature_q.shape[0], feature_q.shape[1]
    feat = feature_q.reshape(bs, C, _S)
    logits = out.reshape(bs, 2, _S)
    tau_arr = jnp.reshape(tau.astype(jnp.float32), (1, 1))

    w = pl.pallas_call(
        _weights_kernel,
        in_specs=[
            pl.BlockSpec((bs, 2, _S), lambda: (0, 0, 0)),
            pl.BlockSpec((1, 1), lambda: (0, 0)),
        ],
        out_specs=pl.BlockSpec((bs, 2, _S), lambda: (0, 0, 0)),
        out_shape=jax.ShapeDtypeStruct((bs, 2, _S), jnp.float32),
    )(logits, tau_arr)

    nc = C // _CB
    protos = pl.pallas_call(
        _pool_kernel,
        grid=(bs, nc),
        compiler_params=pltpu.CompilerParams(
            dimension_semantics=(pltpu.PARALLEL, pltpu.PARALLEL),
        ),
        in_specs=[
            pl.BlockSpec((1, _CB, _S), lambda b, j: (b, j, 0)),
            pl.BlockSpec((1, 2, _S), lambda b, j: (b, 0, 0)),
        ],
        out_specs=pl.BlockSpec((1, 2, _CB), lambda b, j: (b, 0, j)),
        out_shape=jax.ShapeDtypeStruct((bs, 2, C), jnp.float32),
    )(feat, w)

    fg = protos[:, 0, :].reshape(bs, C, 1, 1)
    bg = protos[:, 1, :].reshape(bs, C, 1, 1)
    return fg, bg


def kernel(feature_q, out, tau):
    return _run(feature_q, out, jnp.asarray(tau))


# speedup vs baseline: 1.0911x; 1.0911x over previous
import functools

import jax
import jax.numpy as jnp
from jax.experimental import pallas as pl
from jax.experimental.pallas import tpu as pltpu

_S = 4096
_K = 12
_CB = 256


def _topk_weights(p, iota):
    def body(_, carry):
        pv, accw = carry
        mx = jnp.max(pv, axis=1, keepdims=True)
        eq = pv == mx
        first = jnp.min(jnp.where(eq, iota, _S), axis=1, keepdims=True)
        sel = iota == first
        accw = accw + sel.astype(jnp.float32)
        pv = jnp.where(sel, -jnp.inf, pv)
        return pv, accw

    _, accw = jax.lax.fori_loop(
        0, _K, body, (p, jnp.zeros_like(p)), unroll=True
    )
    return accw


def _weights_kernel(out_ref, tau_ref, w_ref):
    o0 = out_ref[:, 0, :]
    o1 = out_ref[:, 1, :]
    m = jnp.maximum(o0, o1)
    e0 = jnp.exp(o0 - m)
    e1 = jnp.exp(o1 - m)
    s = e0 + e1
    pf = e1 / s
    pb = e0 / s

    tau = tau_ref[0, 0]
    fg_thres = jax.nn.sigmoid(tau)
    bg_thres = 1.0 - fg_thres

    iota = jax.lax.broadcasted_iota(jnp.int32, pf.shape, 1)
    inv_k = jnp.float32(1.0 / _K)

    for cls, p, thres in ((0, pf, fg_thres), (1, pb, bg_thres)):
        mask = (p > thres).astype(jnp.float32)
        cnt = jnp.sum(mask, axis=1, keepdims=True)
        tk = _topk_weights(p, iota)
        safe = jnp.where(cnt > 0, cnt, jnp.float32(1.0))
        w = jnp.where(cnt > 0, mask / safe, tk * inv_k)
        w_ref[:, cls, :] = w


def _pool_kernel(feat_ref, w_ref, out_ref):
    feat = feat_ref[0]
    w = w_ref[0]
    out_ref[0] = jax.lax.dot_general(
        w, feat, (((1,), (1,)), ((), ())),
        preferred_element_type=jnp.float32,
    )


@jax.jit
def _run(feature_q, out, tau):
    bs, C = feature_q.shape[0], feature_q.shape[1]
    feat = feature_q.reshape(bs, C, _S)
    logits = out.reshape(bs, 2, _S)
    tau_arr = jnp.reshape(tau.astype(jnp.float32), (1, 1))

    w = pl.pallas_call(
        _weights_kernel,
        in_specs=[
            pl.BlockSpec((bs, 2, _S), lambda: (0, 0, 0)),
            pl.BlockSpec((1, 1), lambda: (0, 0)),
        ],
        out_specs=pl.BlockSpec((bs, 2, _S), lambda: (0, 0, 0)),
        out_shape=jax.ShapeDtypeStruct((bs, 2, _S), jnp.float32),
    )(logits, tau_arr)

    nc = C // _CB
    protos = pl.pallas_call(
        _pool_kernel,
        grid=(bs, nc),
        compiler_params=pltpu.CompilerParams(
            dimension_semantics=(pltpu.PARALLEL, pltpu.PARALLEL),
        ),
        in_specs=[
            pl.BlockSpec((1, _CB, _S), lambda b, j: (b, j, 0)),
            pl.BlockSpec((1, 2, _S), lambda b, j: (b, 0, 0)),
        ],
        out_specs=pl.BlockSpec((1, 2, _CB), lambda b, j: (b, 0, j)),
        out_shape=jax.ShapeDtypeStruct((bs, 2, C), jnp.float32),
    )(feat, w)

    fg = protos[:, 0, :].reshape(bs, C, 1, 1)
    bg = protos[:, 1, :].reshape(bs, C, 1, 1)
    return fg, bg


def kernel(feature_q, out, tau):
    return _run(feature_q, out, jnp.asarray(tau))


# speedup vs baseline: 1.1307x; 1.0363x over previous
import functools

import jax
import jax.numpy as jnp
from jax.experimental import pallas as pl
from jax.experimental.pallas import tpu as pltpu

_S = 4096
_K = 12
_CB = 512


def _topk_weights(p, iota):
    def body(_, carry):
        pv, accw = carry
        mx = jnp.max(pv, axis=1, keepdims=True)
        eq = pv == mx
        first = jnp.min(jnp.where(eq, iota, _S), axis=1, keepdims=True)
        sel = iota == first
        accw = accw + sel.astype(jnp.float32)
        pv = jnp.where(sel, -jnp.inf, pv)
        return pv, accw

    _, accw = jax.lax.fori_loop(
        0, _K, body, (p, jnp.zeros_like(p)), unroll=True
    )
    return accw


def _weights_kernel(out_ref, tau_ref, w_ref):
    o0 = out_ref[:, 0, :]
    o1 = out_ref[:, 1, :]
    m = jnp.maximum(o0, o1)
    e0 = jnp.exp(o0 - m)
    e1 = jnp.exp(o1 - m)
    s = e0 + e1
    pf = e1 / s
    pb = e0 / s

    tau = tau_ref[0, 0]
    fg_thres = jax.nn.sigmoid(tau)
    bg_thres = 1.0 - fg_thres

    iota = jax.lax.broadcasted_iota(jnp.int32, pf.shape, 1)
    inv_k = jnp.float32(1.0 / _K)

    for cls, p, thres in ((0, pf, fg_thres), (1, pb, bg_thres)):
        mask = (p > thres).astype(jnp.float32)
        cnt = jnp.sum(mask, axis=1, keepdims=True)
        tk = _topk_weights(p, iota)
        safe = jnp.where(cnt > 0, cnt, jnp.float32(1.0))
        w = jnp.where(cnt > 0, mask / safe, tk * inv_k)
        w_ref[:, cls, :] = w


def _pool_kernel(feat_ref, w_ref, out_ref):
    feat = feat_ref[0]
    w = w_ref[0]
    out_ref[0] = jax.lax.dot_general(
        w, feat, (((1,), (1,)), ((), ())),
        preferred_element_type=jnp.float32,
    )


@jax.jit
def _run(feature_q, out, tau):
    bs, C = feature_q.shape[0], feature_q.shape[1]
    feat = feature_q.reshape(bs, C, _S)
    logits = out.reshape(bs, 2, _S)
    tau_arr = jnp.reshape(tau.astype(jnp.float32), (1, 1))

    w = pl.pallas_call(
        _weights_kernel,
        in_specs=[
            pl.BlockSpec((bs, 2, _S), lambda: (0, 0, 0)),
            pl.BlockSpec((1, 1), lambda: (0, 0)),
        ],
        out_specs=pl.BlockSpec((bs, 2, _S), lambda: (0, 0, 0)),
        out_shape=jax.ShapeDtypeStruct((bs, 2, _S), jnp.float32),
    )(logits, tau_arr)

    nc = C // _CB
    protos = pl.pallas_call(
        _pool_kernel,
        grid=(bs, nc),
        compiler_params=pltpu.CompilerParams(
            dimension_semantics=(pltpu.PARALLEL, pltpu.PARALLEL),
        ),
        in_specs=[
            pl.BlockSpec((1, _CB, _S), lambda b, j: (b, j, 0)),
            pl.BlockSpec((1, 2, _S), lambda b, j: (b, 0, 0)),
        ],
        out_specs=pl.BlockSpec((1, 2, _CB), lambda b, j: (b, 0, j)),
        out_shape=jax.ShapeDtypeStruct((bs, 2, C), jnp.float32),
    )(feat, w)

    fg = protos[:, 0, :].reshape(bs, C, 1, 1)
    bg = protos[:, 1, :].reshape(bs, C, 1, 1)
    return fg, bg


def kernel(feature_q, out, tau):
    return _run(feature_q, out, jnp.asarray(tau))
